# baseline (speedup 1.0000x reference)
import functools

import jax
import jax.numpy as jnp
from jax import lax
from jax.experimental import pallas as pl
from jax.experimental.pallas import tpu as pltpu
from jax.experimental.pallas import tpu_sc as plsc

N = 10000
E = 320000
D = 128

NC = 2
NS = 16
NW = NC * NS
EPT = E // NW
CH = 80
NCHUNK = EPT // CH
UNROLL = 6
RCH = 80
NRCH = N // RCH
RITER = (NRCH + NS - 1) // NS
HR = 80
HC = 128


def _sc_aggregate(feat, src, dst):
    mesh = plsc.VectorSubcoreMesh(core_axis_name="c", subcore_axis_name="s")

    @functools.partial(
        pl.kernel,
        out_type=(
            jax.ShapeDtypeStruct((NC, N, D), jnp.float32),
            jax.ShapeDtypeStruct((NC, HR, HC), jnp.float32),
        ),
        mesh=mesh,
        compiler_params=pltpu.CompilerParams(needs_layout_passes=False),
        scratch_types=[
            pltpu.VMEM((CH, D), jnp.float32),
            pltpu.VMEM((CH, D), jnp.float32),
            pltpu.VMEM((CH, D), jnp.float32),
            pltpu.VMEM((CH,), jnp.int32),
            pltpu.VMEM((CH,), jnp.int32),
            pltpu.VMEM((CH,), jnp.int32),
            pltpu.VMEM((CH,), jnp.int32),
            pltpu.VMEM((CH,), jnp.int32),
            pltpu.VMEM((CH,), jnp.int32),
            pltpu.VMEM((HR, HC), jnp.float32),
            pltpu.VMEM((HR,), jnp.int32),
            pltpu.VMEM_SHARED((N, D), jnp.float32),
            pltpu.VMEM_SHARED((HR, HC), jnp.float32),
            pltpu.SemaphoreType.DMA,
            pltpu.SemaphoreType.DMA,
            pltpu.SemaphoreType.DMA,
            pltpu.SemaphoreType.DMA,
            pltpu.SemaphoreType.DMA,
            pltpu.SemaphoreType.DMA,
            pltpu.SemaphoreType.DMA,
            pltpu.SemaphoreType.DMA,
        ],
    )
    def agg_kernel(feat_hbm, src_hbm, dst_hbm, agg_out, deg_out,
                   rows0, rows1, rows2, srcv0, srcv1, dstv0, dstv1,
                   scatv0, scatv1, hist, idrow, acc, dhist,
                   gsem0, gsem1, gsem2, ssem0, ssem1, ssem2, isem0, isem1):
        c = lax.axis_index("c")
        s = lax.axis_index("s")
        wid = c * NS + s
        rows = (rows0, rows1, rows2)
        srcv = (srcv0, srcv1)
        dstv = (dstv0, dstv1)
        scatv = (scatv0, scatv1)
        gsem = (gsem0, gsem1, gsem2)
        ssem = (ssem0, ssem1, ssem2)
        isem = (isem0, isem1)

        zero16 = jnp.zeros((16,), jnp.float32)
        one16 = jnp.ones((16,), jnp.float32)
        iota16 = lax.iota(jnp.int32, 16)

        def zrow(r, carry):
            def zcol(j, carry2):
                rows0[r, pl.ds(j * 16, 16)] = zero16
                return carry2
            return lax.fori_loop(0, D // 16, zcol, carry)
        lax.fori_loop(0, CH, zrow, 0)

        def zhist(r, carry):
            def zcol(j, carry2):
                hist[r, pl.ds(j * 16, 16)] = zero16
                return carry2
            return lax.fori_loop(0, HC // 16, zcol, carry)
        lax.fori_loop(0, HR, zhist, 0)

        for j in range(HR // 16):
            idrow[pl.ds(j * 16, 16)] = iota16 + (j * 16)

        def zero_chunk(k, carry):
            chunk = k * NS + s
            @pl.when(chunk < NRCH)
            def _():
                pltpu.sync_copy(rows0, acc.at[pl.ds(chunk * RCH, RCH)])
            return carry
        lax.fori_loop(0, RITER, zero_chunk, 0)
        @pl.when(s < HR // 8)
        def _():
            pltpu.sync_copy(hist.at[pl.ds(s * 8, 8)], dhist.at[pl.ds(s * 8, 8)])

        plsc.subcore_barrier()

        ebase = wid * EPT

        def issue_idx(chunk, b2):
            off = ebase + chunk * CH
            pltpu.async_copy(src_hbm.at[pl.ds(off, CH)], srcv[b2], isem[b2])
            pltpu.async_copy(dst_hbm.at[pl.ds(off, CH)], dstv[b2], isem[b2])

        def wait_idx(b2):
            pltpu.make_async_copy(
                src_hbm.at[pl.ds(0, CH)], srcv[b2], isem[b2]).wait()
            pltpu.make_async_copy(
                dst_hbm.at[pl.ds(0, CH)], dstv[b2], isem[b2]).wait()

        def issue_gather(b2, b3):
            pltpu.async_copy(feat_hbm.at[srcv[b2]], rows[b3], gsem[b3])

        def wait_gather(b3):
            pltpu.make_async_copy(
                feat_hbm.at[srcv[0]], rows[b3], gsem[b3]).wait()

        def issue_scatter(b2, b3):
            pltpu.async_copy(rows[b3], acc.at[scatv[b2]], ssem[b3], add=True)

        def wait_scatter(b3):
            pltpu.make_async_copy(
                rows[b3], acc.at[scatv[0]], ssem[b3]).wait()

        issue_idx(0, 0)
        issue_idx(1, 1)
        wait_idx(0)
        issue_gather(0, 0)

        @pl.loop(0, NCHUNK + 1, step=UNROLL)
        def edge_loop(g):
            for u in range(UNROLL):
                chunk = g + u
                b3 = u % 3
                b2 = u % 2
                n3 = (u + 1) % 3
                n2 = (u + 1) % 2
                @pl.when(chunk < NCHUNK)
                def _():
                    wait_gather(b3)
                    @pl.when(chunk + 1 < NCHUNK)
                    def _():
                        @pl.when(chunk >= 2)
                        def _():
                            wait_scatter(n3)
                        wait_idx(n2)
                        issue_gather(n2, n3)
                    for j in range(CH // 16):
                        dv = dstv[b2][pl.ds(j * 16, 16)]
                        scatv[b2][pl.ds(j * 16, 16)] = dv
                        plsc.addupdate_scatter(
                            hist, [dv // HC, dv % HC], one16)
                    @pl.when(chunk + 2 < NCHUNK)
                    def _():
                        issue_idx(chunk + 2, b2)
                    issue_scatter(b2, b3)

        wait_scatter(0)
        wait_scatter(1)
        wait_scatter(2)

        pltpu.sync_copy(hist, dhist.at[idrow], add=True)

        plsc.subcore_barrier()

        @pl.when(s < HR // 8)
        def _():
            pltpu.sync_copy(dhist.at[pl.ds(s * 8, 8)], hist.at[pl.ds(0, 8)])
            pltpu.sync_copy(hist.at[pl.ds(0, 8)],
                            deg_out.at[c, pl.ds(s * 8, 8)])

        pltpu.async_copy(acc.at[pl.ds(s * RCH, RCH)], rows0, gsem0)

        @pl.loop(0, RITER, step=2)
        def out_loop(k):
            for b in range(2):
                kk = k + b
                chunk = kk * NS + s
                @pl.when(chunk < NRCH)
                def _():
                    pltpu.make_async_copy(
                        acc.at[pl.ds(0, RCH)], rows[b], gsem[b]).wait()
                    nxt = (kk + 1) * NS + s
                    @pl.when(nxt < NRCH)
                    def _():
                        pltpu.async_copy(acc.at[pl.ds(nxt * RCH, RCH)],
                                         rows[1 - b], gsem[1 - b])
                    pltpu.sync_copy(rows[b],
                                    agg_out.at[c, pl.ds(chunk * RCH, RCH)])

    return agg_kernel(feat, src, dst)


BR = 1000


def _combine_body(feat_ref, wst_ref, wnt_ref, b_ref, agg_ref, deg_ref, out_ref):
    deg = jnp.maximum(deg_ref[0] + deg_ref[1], 1.0)
    m = (agg_ref[0] + agg_ref[1]) / deg
    out_ref[...] = (
        jnp.dot(feat_ref[...], wst_ref[...], preferred_element_type=jnp.float32)
        + jnp.dot(m, wnt_ref[...], preferred_element_type=jnp.float32)
        + b_ref[...]
    )


def kernel(feat, edge_index, W_self, W_neigh, b_neigh):
    src = edge_index[0].astype(jnp.int32)
    dst = edge_index[1].astype(jnp.int32)

    agg, deg_hist = _sc_aggregate(feat, src, dst)
    deg = deg_hist.reshape(NC, HR * HC)[:, :N]

    h = pl.pallas_call(
        _combine_body,
        grid=(N // BR,),
        in_specs=[
            pl.BlockSpec((BR, D), lambda i: (i, 0)),
            pl.BlockSpec((D, D), lambda i: (0, 0)),
            pl.BlockSpec((D, D), lambda i: (0, 0)),
            pl.BlockSpec((1, D), lambda i: (0, 0)),
            pl.BlockSpec((NC, BR, D), lambda i: (0, i, 0)),
            pl.BlockSpec((NC, BR, 1), lambda i: (0, i, 0)),
        ],
        out_specs=pl.BlockSpec((BR, D), lambda i: (i, 0)),
        out_shape=jax.ShapeDtypeStruct((N, D), jnp.float32),
    )(feat, W_self.T, W_neigh.T, b_neigh.reshape(1, D), agg,
      deg.reshape(NC, N, 1))

    return h


# speedup vs baseline: 1.1588x; 1.1588x over previous
import functools

import jax
import jax.numpy as jnp
from jax import lax
from jax.experimental import pallas as pl
from jax.experimental.pallas import tpu as pltpu
from jax.experimental.pallas import tpu_sc as plsc

N = 10000
E = 320000
D = 128

NC = 2
NS = 16
NW = NC * NS
CH = 128
TOTCH = E // CH
KITER = (TOTCH + NW - 1) // NW + 1
RCH = 80
NRCH = N // RCH
RITER = (NRCH + NS - 1) // NS
HR = 80
HC = 128


def _sc_aggregate(feat, src, dst):
    mesh = plsc.VectorSubcoreMesh(core_axis_name="c", subcore_axis_name="s")

    @functools.partial(
        pl.kernel,
        out_type=(
            jax.ShapeDtypeStruct((NC, N, D), jnp.float32),
            jax.ShapeDtypeStruct((NC, HR, HC), jnp.float32),
        ),
        mesh=mesh,
        compiler_params=pltpu.CompilerParams(needs_layout_passes=False),
        scratch_types=[
            pltpu.VMEM((CH, D), jnp.float32),
            pltpu.VMEM((CH, D), jnp.float32),
            pltpu.VMEM((CH,), jnp.int32),
            pltpu.VMEM((CH,), jnp.int32),
            pltpu.VMEM((CH,), jnp.int32),
            pltpu.VMEM((CH,), jnp.int32),
            pltpu.VMEM((CH,), jnp.int32),
            pltpu.VMEM((CH,), jnp.int32),
            pltpu.VMEM((HR, HC), jnp.float32),
            pltpu.VMEM((HR,), jnp.int32),
            pltpu.VMEM_SHARED((N, D), jnp.float32),
            pltpu.VMEM_SHARED((HR, HC), jnp.float32),
            pltpu.SemaphoreType.DMA,
            pltpu.SemaphoreType.DMA,
            pltpu.SemaphoreType.DMA,
            pltpu.SemaphoreType.DMA,
            pltpu.SemaphoreType.DMA,
            pltpu.SemaphoreType.DMA,
        ],
    )
    def agg_kernel(feat_hbm, src_hbm, dst_hbm, agg_out, deg_out,
                   rows0, rows1, srcv0, srcv1, dstv0, dstv1,
                   scatv0, scatv1, hist, idrow, acc, dhist,
                   gsem0, gsem1, ssem0, ssem1, isem0, isem1):
        c = lax.axis_index("c")
        s = lax.axis_index("s")
        wid = c * NS + s
        rows = (rows0, rows1)
        srcv = (srcv0, srcv1)
        dstv = (dstv0, dstv1)
        scatv = (scatv0, scatv1)
        gsem = (gsem0, gsem1)
        ssem = (ssem0, ssem1)
        isem = (isem0, isem1)

        zero16 = jnp.zeros((16,), jnp.float32)
        one16 = jnp.ones((16,), jnp.float32)
        iota16 = lax.iota(jnp.int32, 16)

        def zrow(r, carry):
            def zcol(j, carry2):
                rows0[r, pl.ds(j * 16, 16)] = zero16
                return carry2
            return lax.fori_loop(0, D // 16, zcol, carry)
        lax.fori_loop(0, RCH, zrow, 0)

        def zhist(r, carry):
            def zcol(j, carry2):
                hist[r, pl.ds(j * 16, 16)] = zero16
                return carry2
            return lax.fori_loop(0, HC // 16, zcol, carry)
        lax.fori_loop(0, HR, zhist, 0)

        for j in range(HR // 16):
            idrow[pl.ds(j * 16, 16)] = iota16 + (j * 16)

        def zero_chunk(k, carry):
            chunk = k * NS + s
            @pl.when(chunk < NRCH)
            def _():
                pltpu.sync_copy(rows0.at[pl.ds(0, RCH)],
                                acc.at[pl.ds(chunk * RCH, RCH)])
            return carry
        lax.fori_loop(0, RITER, zero_chunk, 0)
        @pl.when(s < HR // 8)
        def _():
            pltpu.sync_copy(hist.at[pl.ds(s * 8, 8)], dhist.at[pl.ds(s * 8, 8)])

        plsc.subcore_barrier()

        def cid(k):
            return wid + k * NW

        def issue_idx(k, b):
            off = cid(k) * CH
            pltpu.async_copy(src_hbm.at[pl.ds(off, CH)], srcv[b], isem[b])
            pltpu.async_copy(dst_hbm.at[pl.ds(off, CH)], dstv[b], isem[b])

        def wait_idx(b):
            pltpu.make_async_copy(
                src_hbm.at[pl.ds(0, CH)], srcv[b], isem[b]).wait()
            pltpu.make_async_copy(
                dst_hbm.at[pl.ds(0, CH)], dstv[b], isem[b]).wait()

        def issue_gather(b):
            pltpu.async_copy(feat_hbm.at[srcv[b]], rows[b], gsem[b])

        def wait_gather(b):
            pltpu.make_async_copy(
                feat_hbm.at[srcv[0]], rows[b], gsem[b]).wait()

        def issue_scatter(b):
            pltpu.async_copy(rows[b], acc.at[scatv[b]], ssem[b], add=True)

        def wait_scatter(b):
            pltpu.make_async_copy(
                rows[b], acc.at[scatv[0]], ssem[b]).wait()

        issue_idx(0, 0)
        issue_idx(1, 1)
        wait_idx(0)
        issue_gather(0)

        @pl.loop(0, KITER, step=2)
        def edge_loop(g):
            for b in range(2):
                k = g + b
                nb = 1 - b
                @pl.when(cid(k) < TOTCH)
                def _():
                    wait_gather(b)
                    @pl.when(cid(k + 1) < TOTCH)
                    def _():
                        @pl.when(k >= 1)
                        def _():
                            wait_scatter(nb)
                        wait_idx(nb)
                        issue_gather(nb)
                    for j in range(CH // 16):
                        dv = dstv[b][pl.ds(j * 16, 16)]
                        scatv[b][pl.ds(j * 16, 16)] = dv
                        plsc.addupdate_scatter(
                            hist, [dv // HC, dv % HC], one16)
                    @pl.when(cid(k + 2) < TOTCH)
                    def _():
                        issue_idx(k + 2, b)
                    issue_scatter(b)

        wait_scatter(0)
        wait_scatter(1)

        pltpu.sync_copy(hist, dhist.at[idrow], add=True)

        plsc.subcore_barrier()

        @pl.when(s < HR // 8)
        def _():
            pltpu.sync_copy(dhist.at[pl.ds(s * 8, 8)], hist.at[pl.ds(0, 8)])
            pltpu.sync_copy(hist.at[pl.ds(0, 8)],
                            deg_out.at[c, pl.ds(s * 8, 8)])

        pltpu.async_copy(acc.at[pl.ds(s * RCH, RCH)],
                         rows0.at[pl.ds(0, RCH)], gsem0)

        @pl.loop(0, RITER, step=2)
        def out_loop(kk0):
            for b in range(2):
                kk = kk0 + b
                chunk = kk * NS + s
                @pl.when(chunk < NRCH)
                def _():
                    pltpu.make_async_copy(
                        acc.at[pl.ds(0, RCH)],
                        rows[b].at[pl.ds(0, RCH)], gsem[b]).wait()
                    nxt = (kk + 1) * NS + s
                    @pl.when(nxt < NRCH)
                    def _():
                        pltpu.async_copy(acc.at[pl.ds(nxt * RCH, RCH)],
                                         rows[1 - b].at[pl.ds(0, RCH)],
                                         gsem[1 - b])
                    pltpu.sync_copy(rows[b].at[pl.ds(0, RCH)],
                                    agg_out.at[c, pl.ds(chunk * RCH, RCH)])

    return agg_kernel(feat, src, dst)


BR = 1000


def _combine_body(feat_ref, wst_ref, wnt_ref, b_ref, agg_ref, deg_ref, out_ref):
    deg = jnp.maximum(deg_ref[...], 1.0)
    m = (agg_ref[0] + agg_ref[1]) / deg
    out_ref[...] = (
        jnp.dot(feat_ref[...], wst_ref[...], preferred_element_type=jnp.float32)
        + jnp.dot(m, wnt_ref[...], preferred_element_type=jnp.float32)
        + b_ref[...]
    )


def kernel(feat, edge_index, W_self, W_neigh, b_neigh):
    src = edge_index[0].astype(jnp.int32)
    dst = edge_index[1].astype(jnp.int32)

    agg, deg_hist = _sc_aggregate(feat, src, dst)
    deg = (deg_hist[0] + deg_hist[1]).reshape(HR * HC)[:N]

    h = pl.pallas_call(
        _combine_body,
        grid=(N // BR,),
        in_specs=[
            pl.BlockSpec((BR, D), lambda i: (i, 0)),
            pl.BlockSpec((D, D), lambda i: (0, 0)),
            pl.BlockSpec((D, D), lambda i: (0, 0)),
            pl.BlockSpec((1, D), lambda i: (0, 0)),
            pl.BlockSpec((NC, BR, D), lambda i: (0, i, 0)),
            pl.BlockSpec((BR, 1), lambda i: (i, 0)),
        ],
        out_specs=pl.BlockSpec((BR, D), lambda i: (i, 0)),
        out_shape=jax.ShapeDtypeStruct((N, D), jnp.float32),
    )(feat, W_self.T, W_neigh.T, b_neigh.reshape(1, D), agg,
      deg.reshape(N, 1))

    return h
